# Initial kernel scaffold; baseline (speedup 1.0000x reference)
import jax
import jax.numpy as jnp
from jax.experimental import pallas as pl


def kernel(x, W_orig, b_orig, W_pol, b_pol):
    raise NotImplementedError("write your pallas kernel here")


# speedup vs baseline: 3.2870x; 3.2870x over previous
import functools
import math

import jax
import jax.numpy as jnp
from jax.experimental import pallas as pl
from jax.experimental.pallas import tpu as pltpu

QUANT = 0.9


def _mm1_kernel(x_ref, w_ref, b_ref, y_ref):
    acc = jnp.dot(
        x_ref[...],
        w_ref[...],
        preferred_element_type=jnp.float32,
    )
    y_ref[...] = acc + b_ref[...][None, :]


def _matmul(x, w, b, tn):
    B, D = x.shape
    D2, F = w.shape
    nn = F // tn
    return pl.pallas_call(
        _mm1_kernel,
        grid=(nn,),
        in_specs=[
            pl.BlockSpec((B, D), lambda n: (0, 0)),
            pl.BlockSpec((D, tn), lambda n: (0, n)),
            pl.BlockSpec((tn,), lambda n: (n,)),
        ],
        out_specs=pl.BlockSpec((B, tn), lambda n: (0, n)),
        out_shape=jax.ShapeDtypeStruct((B, F), jnp.float32),
        compiler_params=pltpu.CompilerParams(
            dimension_semantics=("arbitrary",),
        ),
    )(x, w, b)


def _mask_kernel(k_th, y_ref, z_ref, o_ref):
    z = z_ref[...]
    u = jax.lax.bitcast_convert_type(z, jnp.int32)
    key = jnp.where(u >= 0, u, jnp.int32(-2147483648) - u)
    B = z.shape[0]
    lo = jnp.full((B, 1), -(2**31), jnp.int32)
    hi = jnp.full((B, 1), 2**31 - 1, jnp.int32)

    def body(_, carry):
        lo, hi = carry
        mid = (lo >> 1) + (hi >> 1) + (lo & hi & 1)
        cnt = jnp.sum((key <= mid).astype(jnp.int32), axis=1, keepdims=True)
        ge = cnt >= k_th
        lo = jnp.where(ge, lo, mid + 1)
        hi = jnp.where(ge, mid, hi)
        return lo, hi

    lo, hi = jax.lax.fori_loop(0, 32, body, (lo, hi))
    thr = lo
    o_ref[...] = jnp.where(key > thr, y_ref[...], 0.0)


def _mask_stage(y, z, k_th):
    B, F = y.shape
    return pl.pallas_call(
        functools.partial(_mask_kernel, k_th),
        in_specs=[
            pl.BlockSpec((B, F), lambda: (0, 0)),
            pl.BlockSpec((B, F), lambda: (0, 0)),
        ],
        out_specs=pl.BlockSpec((B, F), lambda: (0, 0)),
        out_shape=jax.ShapeDtypeStruct((B, F), jnp.float32),
    )(y, z)


@jax.jit
def kernel(x, W_orig, b_orig, W_pol, b_pol):
    F = W_pol.shape[1]
    k_th = int(max(1, min(F, 1 + math.floor(QUANT * (F - 1)))))
    y = _matmul(x, W_orig, b_orig, tn=512)
    z = _matmul(y, W_pol, b_pol, tn=256)
    return _mask_stage(y, z, k_th)


# speedup vs baseline: 3.5016x; 1.0653x over previous
import functools
import math

import jax
import jax.numpy as jnp
from jax.experimental import pallas as pl
from jax.experimental.pallas import tpu as pltpu

QUANT = 0.9
TA = 512
TB = 256


def _fused_kernel(
    na, nb, k_th,
    x_ref, wo_ref, bo_ref, wp_ref, bp_ref,
    o_ref,
    y_sc, key_sc,
):
    i = pl.program_id(0)

    @pl.when(i < na)
    def _phase_a():
        y_blk = jnp.dot(
            x_ref[...], wo_ref[...], preferred_element_type=jnp.float32
        ) + bo_ref[...][None, :]
        y_sc[:, pl.ds(i * TA, TA)] = y_blk

    @pl.when(i >= na)
    def _phase_b():
        j = i - na
        z_blk = jnp.dot(
            y_sc[...], wp_ref[...], preferred_element_type=jnp.float32
        ) + bp_ref[...][None, :]
        u = jax.lax.bitcast_convert_type(z_blk, jnp.int32)
        key_sc[:, pl.ds(j * TB, TB)] = jnp.where(
            u >= 0, u, jnp.int32(-(2**31)) - u
        )

    @pl.when(i == na + nb - 1)
    def _epilogue():
        B = o_ref.shape[0]
        lo = jnp.full((B, 1), -(2**31), jnp.int32)
        hi = jnp.full((B, 1), 2**31 - 1, jnp.int32)

        def body(_, carry):
            lo, hi = carry
            mid = (lo >> 1) + (hi >> 1) + (lo & hi & 1)
            cnt = jnp.sum(
                (key_sc[...] <= mid).astype(jnp.int32), axis=1, keepdims=True
            )
            ge = cnt >= k_th
            lo = jnp.where(ge, lo, mid + 1)
            hi = jnp.where(ge, mid, hi)
            return lo, hi

        lo, hi = jax.lax.fori_loop(0, 32, body, (lo, hi))
        o_ref[...] = jnp.where(key_sc[...] > lo, y_sc[...], 0.0)


@jax.jit
def kernel(x, W_orig, b_orig, W_pol, b_pol):
    B, D = x.shape
    F = W_pol.shape[1]
    k_th = int(max(1, min(F, 1 + math.floor(QUANT * (F - 1)))))
    na = F // TA
    nb = F // TB

    return pl.pallas_call(
        functools.partial(_fused_kernel, na, nb, k_th),
        grid=(na + nb,),
        in_specs=[
            pl.BlockSpec((B, D), lambda i: (0, 0)),
            pl.BlockSpec((D, TA), lambda i: (0, jnp.minimum(i, na - 1))),
            pl.BlockSpec((TA,), lambda i: (jnp.minimum(i, na - 1),)),
            pl.BlockSpec((F, TB), lambda i: (0, jnp.maximum(0, i - na))),
            pl.BlockSpec((TB,), lambda i: (jnp.maximum(0, i - na),)),
        ],
        out_specs=pl.BlockSpec((B, F), lambda i: (0, 0)),
        out_shape=jax.ShapeDtypeStruct((B, F), jnp.float32),
        scratch_shapes=[
            pltpu.VMEM((B, F), jnp.float32),
            pltpu.VMEM((B, F), jnp.int32),
        ],
        compiler_params=pltpu.CompilerParams(
            dimension_semantics=("arbitrary",),
        ),
    )(x, W_orig, b_orig, W_pol, b_pol)
